# baseline (speedup 1.0000x reference)
import jax
import jax.numpy as jnp
from jax import lax
from jax.experimental import pallas as pl
from jax.experimental.pallas import tpu as pltpu
from jax.experimental.pallas import tpu_sc as plsc

N_NODES = 10000
N_PAD = 10240
N_EDGES = 160000
HID = 64
L = 16
NC = 2
NS = 16
NW = NC * NS
NVB = N_NODES // L
NVB_PAD = N_PAD // L

EBLK = 1280
NBLK = N_EDGES // EBLK
BPB = EBLK // L

_NB = 2048


def _dense_body(x_ref, w_ref, b_ref, out_ref):
    acc = lax.dot_general(w_ref[...], x_ref[...], (((1,), (1,)), ((), ())),
                          preferred_element_type=jnp.float32)
    out_ref[...] = acc + b_ref[...]


def _dense_transposed(x, W1, b1):
    n, k = x.shape
    d = W1.shape[0]
    return pl.pallas_call(
        _dense_body,
        grid=(N_PAD // _NB,),
        in_specs=[
            pl.BlockSpec((_NB, k), lambda i: (i, 0)),
            pl.BlockSpec((d, k), lambda i: (0, 0)),
            pl.BlockSpec((d, 1), lambda i: (0, 0)),
        ],
        out_specs=pl.BlockSpec((d, _NB), lambda i: (0, i)),
        out_shape=jax.ShapeDtypeStruct((d, N_PAD), jnp.float32),
    )(x, W1, b1[:, None])


def _rsqrt16(x):
    i = plsc.bitcast(x, jnp.int32)
    y = plsc.bitcast(jnp.int32(0x5F3759DF) - (i >> 1), jnp.float32)
    for _ in range(3):
        y = y * (1.5 - 0.5 * x * y * y)
    return y


_CBITS = 14
_CMASK = (1 << _CBITS) - 1


_EH = EBLK // 2


def _edge_stage_start(pk2d, vals2d, pb, vb, sems, blk, slot):
    boff = slot * EBLK
    pltpu.async_copy(pk2d.at[blk, pl.ds(0, _EH)],
                     pb.at[pl.ds(boff, _EH)], sems.at[slot, 0])
    pltpu.async_copy(pk2d.at[blk, pl.ds(_EH, _EH)],
                     pb.at[pl.ds(boff + _EH, _EH)], sems.at[slot, 1])
    pltpu.async_copy(vals2d.at[blk, pl.ds(0, _EH)],
                     vb.at[pl.ds(boff, _EH)], sems.at[slot, 2])
    pltpu.async_copy(vals2d.at[blk, pl.ds(_EH, _EH)],
                     vb.at[pl.ds(boff + _EH, _EH)], sems.at[slot, 3])


def _edge_stage_wait(pk2d, vals2d, pb, vb, sems, blk, slot):
    boff = slot * EBLK
    pltpu.make_async_copy(pk2d.at[blk, pl.ds(0, _EH)],
                          pb.at[pl.ds(boff, _EH)], sems.at[slot, 0]).wait()
    pltpu.make_async_copy(pk2d.at[blk, pl.ds(_EH, _EH)],
                          pb.at[pl.ds(boff + _EH, _EH)], sems.at[slot, 1]).wait()
    pltpu.make_async_copy(vals2d.at[blk, pl.ds(0, _EH)],
                          vb.at[pl.ds(boff, _EH)], sems.at[slot, 2]).wait()
    pltpu.make_async_copy(vals2d.at[blk, pl.ds(_EH, _EH)],
                          vb.at[pl.ds(boff + _EH, _EH)], sems.at[slot, 3]).wait()


def _spmm_bn_body(hT, pk2d, vals2d, params, yp_out,
                  h0, h1, a0, a1, ybuf, pb, vb, pv, sems, hsems):
    c = lax.axis_index("c")
    s = lax.axis_index("s")
    wid = s * NC + c
    d0 = wid * 2

    NH = N_PAD // 2
    pltpu.async_copy(hT.at[d0, pl.ds(0, NH)], h0.at[pl.ds(0, NH)],
                     hsems.at[0])
    pltpu.async_copy(hT.at[d0, pl.ds(NH, NH)], h0.at[pl.ds(NH, NH)],
                     hsems.at[1])
    pltpu.async_copy(hT.at[d0 + 1, pl.ds(0, NH)], h1.at[pl.ds(0, NH)],
                     hsems.at[2])
    pltpu.async_copy(hT.at[d0 + 1, pl.ds(NH, NH)], h1.at[pl.ds(NH, NH)],
                     hsems.at[3])
    _edge_stage_start(pk2d, vals2d, pb, vb, sems, 0, 0)
    pltpu.sync_copy(params, pv)

    zero = jnp.zeros((L,), jnp.float32)

    @plsc.parallel_loop(0, NVB_PAD)
    def _(j):
        sl = pl.ds(j * L, L)
        a0[sl] = zero
        a1[sl] = zero
        ybuf[sl] = zero

    pltpu.make_async_copy(hT.at[d0, pl.ds(0, NH)], h0.at[pl.ds(0, NH)],
                          hsems.at[0]).wait()
    pltpu.make_async_copy(hT.at[d0, pl.ds(NH, NH)], h0.at[pl.ds(NH, NH)],
                          hsems.at[1]).wait()
    pltpu.make_async_copy(hT.at[d0 + 1, pl.ds(0, NH)], h1.at[pl.ds(0, NH)],
                          hsems.at[2]).wait()
    pltpu.make_async_copy(hT.at[d0 + 1, pl.ds(NH, NH)], h1.at[pl.ds(NH, NH)],
                          hsems.at[3]).wait()

    def chunk_body(ci, carry):
        slot = jnp.bitwise_and(ci, 1)

        @pl.when(ci + 1 < NBLK)
        def _():
            _edge_stage_start(pk2d, vals2d, pb, vb, sems, ci + 1, 1 - slot)
        _edge_stage_wait(pk2d, vals2d, pb, vb, sems, ci, slot)
        boff = slot * EBLK

        @plsc.parallel_loop(0, BPB, unroll=4)
        def _(k):
            sl = pl.ds(boff + k * L, L)
            pk = pb[sl]
            cidx = jnp.bitwise_and(pk, _CMASK)
            ridx = pk >> _CBITS
            v = vb[sl]
            g0 = plsc.load_gather(h0, [cidx]) * v
            plsc.addupdate_scatter(a0, [ridx], g0)
            g1 = plsc.load_gather(h1, [cidx]) * v
            plsc.addupdate_scatter(a1, [ridx], g1)
        return carry
    lax.fori_loop(0, NBLK, chunk_body, 0)

    def stat(acc):
        def sb(j, carry):
            v = acc[pl.ds(j * L, L)]
            return (carry[0] + v, carry[1] + v * v)
        sv, qv = lax.fori_loop(0, NVB, sb, (zero, zero))
        return jnp.sum(sv), jnp.sum(qv)

    inv_n = 1.0 / N_NODES
    s0, q0 = stat(a0)
    s1, q1 = stat(a1)
    m0 = s0 * inv_n
    m1 = s1 * inv_n
    v0 = q0 * inv_n - m0 * m0
    v1 = q1 * inv_n - m1 * m1

    idx0 = jnp.full((L,), d0, jnp.int32)
    idx1 = jnp.full((L,), d0 + 1, jnp.int32)
    gam0 = plsc.load_gather(pv, [idx0])
    gam1 = plsc.load_gather(pv, [idx1])
    bet0 = plsc.load_gather(pv, [idx0 + HID])
    bet1 = plsc.load_gather(pv, [idx1 + HID])
    w20 = plsc.load_gather(pv, [idx0 + 2 * HID])
    w21 = plsc.load_gather(pv, [idx1 + 2 * HID])
    b2v = pv[pl.ds(3 * HID, L)]

    eps = 1e-5
    inv0 = _rsqrt16(jnp.full((L,), v0) + eps) * gam0
    inv1 = _rsqrt16(jnp.full((L,), v1) + eps) * gam1
    sh0 = bet0 - jnp.full((L,), m0) * inv0
    sh1 = bet1 - jnp.full((L,), m1) * inv1
    b2add = b2v * jnp.where(wid == 0, 1.0, 0.0)

    @plsc.parallel_loop(0, NVB, unroll=4)
    def _(j):
        sl = pl.ds(j * L, L)
        t0 = jnp.maximum(a0[sl] * inv0 + sh0, 0.0)
        t1 = jnp.maximum(a1[sl] * inv1 + sh1, 0.0)
        ybuf[sl] = t0 * w20 + t1 * w21 + b2add

    pltpu.sync_copy(ybuf, yp_out.at[wid])


def _spmm_bn(hT, pk2d, vals2d, params):
    mesh = plsc.VectorSubcoreMesh(core_axis_name="c", subcore_axis_name="s")
    f = pl.kernel(
        _spmm_bn_body,
        out_type=jax.ShapeDtypeStruct((NW, N_PAD), jnp.float32),
        mesh=mesh,
        compiler_params=pltpu.CompilerParams(needs_layout_passes=False),
        scratch_types=[
            pltpu.VMEM((N_PAD,), jnp.float32),
            pltpu.VMEM((N_PAD,), jnp.float32),
            pltpu.VMEM((N_PAD,), jnp.float32),
            pltpu.VMEM((N_PAD,), jnp.float32),
            pltpu.VMEM((N_PAD,), jnp.float32),
            pltpu.VMEM((2 * EBLK,), jnp.int32),
            pltpu.VMEM((2 * EBLK,), jnp.float32),
            pltpu.VMEM((256,), jnp.float32),
            pltpu.SemaphoreType.DMA((2, 4)),
            pltpu.SemaphoreType.DMA((4,)),
        ],
    )
    return f(hT, pk2d, vals2d, params)


_RNG = N_PAD // NS


def _spmm2_body(yp, pk2d, vals2d, out_hbm,
                yv, acc, tmp, ys, pb, vb, ysh, osl, sems, rsem):
    c = lax.axis_index("c")
    s = lax.axis_index("s")
    zero = jnp.zeros((L,), jnp.float32)
    nb = _RNG // L

    def reduce_slots(src, nslots, off):
        @plsc.parallel_loop(0, nb)
        def _(j):
            ys[pl.ds(j * L, L)] = zero

        def start(t):
            toff = jnp.bitwise_and(t, 1) * _RNG
            pltpu.async_copy(src.at[t, pl.ds(off, _RNG)],
                             tmp.at[pl.ds(toff, _RNG)], rsem.at[jnp.bitwise_and(t, 1)])

        def wait(t):
            toff = jnp.bitwise_and(t, 1) * _RNG
            pltpu.make_async_copy(src.at[t, pl.ds(off, _RNG)],
                                  tmp.at[pl.ds(toff, _RNG)],
                                  rsem.at[jnp.bitwise_and(t, 1)]).wait()

        start(0)

        def tb(t, carry):
            @pl.when(t + 1 < nslots)
            def _():
                start(t + 1)
            wait(t)
            toff = jnp.bitwise_and(t, 1) * _RNG

            @plsc.parallel_loop(0, nb, unroll=4)
            def _(j):
                sl = pl.ds(j * L, L)
                ys[sl] = ys[sl] + tmp[pl.ds(toff + j * L, L)]
            return carry
        lax.fori_loop(0, nslots, tb, 0)

    @pl.when(c == 0)
    def _():
        off = pl.multiple_of(s * _RNG, 128)
        reduce_slots(yp, NW, off)
        pltpu.sync_copy(ys, ysh.at[pl.ds(off, _RNG)])
        plsc.subcore_barrier()
        pltpu.sync_copy(ysh, yv)

        @plsc.parallel_loop(0, NVB_PAD)
        def _(j):
            acc[pl.ds(j * L, L)] = zero

        nrounds = (NBLK + NS - 1) // NS

        def start_blk(ci, slot):
            _edge_stage_start(pk2d, vals2d, pb, vb, sems, ci * NS + s, slot)

        def wait_blk(ci, slot):
            _edge_stage_wait(pk2d, vals2d, pb, vb, sems, ci * NS + s, slot)

        start_blk(0, 0)

        def chunk_body(ci, carry):
            blk = ci * NS + s
            slot = jnp.bitwise_and(ci, 1)

            @pl.when(blk < NBLK)
            def _():
                @pl.when(blk + NS < NBLK)
                def _():
                    start_blk(ci + 1, 1 - slot)
                wait_blk(ci, slot)
                boff = slot * EBLK

                @plsc.parallel_loop(0, BPB, unroll=4)
                def _(k):
                    sl = pl.ds(boff + k * L, L)
                    pk = pb[sl]
                    g = plsc.load_gather(yv, [jnp.bitwise_and(pk, _CMASK)]) * vb[sl]
                    plsc.addupdate_scatter(acc, [pk >> _CBITS], g)
            return carry
        lax.fori_loop(0, nrounds, chunk_body, 0)

        pltpu.sync_copy(acc, osl.at[s])
        plsc.subcore_barrier()

        reduce_slots(osl, NS, off)
        pltpu.sync_copy(ys, out_hbm.at[pl.ds(off, _RNG)])


def _spmm_scalar(yp, pk2d, vals2d):
    mesh = plsc.VectorSubcoreMesh(core_axis_name="c", subcore_axis_name="s")
    f = pl.kernel(
        _spmm2_body,
        out_type=jax.ShapeDtypeStruct((N_PAD,), jnp.float32),
        mesh=mesh,
        compiler_params=pltpu.CompilerParams(needs_layout_passes=False),
        scratch_types=[
            pltpu.VMEM((N_PAD,), jnp.float32),
            pltpu.VMEM((N_PAD,), jnp.float32),
            pltpu.VMEM((2 * _RNG,), jnp.float32),
            pltpu.VMEM((_RNG,), jnp.float32),
            pltpu.VMEM((2 * EBLK,), jnp.int32),
            pltpu.VMEM((2 * EBLK,), jnp.float32),
            pltpu.VMEM_SHARED((N_PAD,), jnp.float32),
            pltpu.VMEM_SHARED((NS, N_PAD), jnp.float32),
            pltpu.SemaphoreType.DMA((2, 4)),
            pltpu.SemaphoreType.DMA((2,)),
        ],
    )
    return f(yp, pk2d, vals2d)


def kernel(x, A_indices, A_values, W1, b1, gamma, beta, W2, b2):
    rows = A_indices[0].astype(jnp.int32)
    cols = A_indices[1].astype(jnp.int32)
    pk2d = ((rows << _CBITS) | cols).reshape(NBLK, EBLK)
    vals2d = A_values.astype(jnp.float32).reshape(NBLK, EBLK)
    params = jnp.concatenate([
        gamma.astype(jnp.float32),
        beta.astype(jnp.float32),
        W2.reshape(-1).astype(jnp.float32),
        jnp.full((L,), b2[0], jnp.float32),
        jnp.zeros((256 - 3 * HID - L,), jnp.float32),
    ])
    hT = _dense_transposed(x, W1, b1)
    yp = _spmm_bn(hT, pk2d, vals2d, params)
    out = _spmm_scalar(yp, pk2d, vals2d)
    return out[:N_NODES]


# speedup vs baseline: 1.0239x; 1.0239x over previous
import jax
import jax.numpy as jnp
from jax import lax
from jax.experimental import pallas as pl
from jax.experimental.pallas import tpu as pltpu
from jax.experimental.pallas import tpu_sc as plsc

N_NODES = 10000
N_PAD = 10240
N_EDGES = 160000
HID = 64
L = 16
NC = 2
NS = 16
NW = NC * NS
NVB = N_NODES // L
NVB_PAD = N_PAD // L

EBLK = 1280
NBLK = N_EDGES // EBLK
BPB = EBLK // L

_NB = 2048


def _dense_body(x_ref, w_ref, b_ref, out_ref):
    acc = lax.dot_general(w_ref[...], x_ref[...], (((1,), (1,)), ((), ())),
                          preferred_element_type=jnp.float32)
    out_ref[...] = acc + b_ref[...]


def _dense_transposed(x, W1, b1):
    n, k = x.shape
    d = W1.shape[0]
    return pl.pallas_call(
        _dense_body,
        grid=(N_PAD // _NB,),
        in_specs=[
            pl.BlockSpec((_NB, k), lambda i: (i, 0)),
            pl.BlockSpec((d, k), lambda i: (0, 0)),
            pl.BlockSpec((d, 1), lambda i: (0, 0)),
        ],
        out_specs=pl.BlockSpec((d, _NB), lambda i: (0, i)),
        out_shape=jax.ShapeDtypeStruct((d, N_PAD), jnp.float32),
    )(x, W1, b1[:, None])


def _rsqrt16(x):
    i = plsc.bitcast(x, jnp.int32)
    y = plsc.bitcast(jnp.int32(0x5F3759DF) - (i >> 1), jnp.float32)
    for _ in range(3):
        y = y * (1.5 - 0.5 * x * y * y)
    return y


_CBITS = 14
_CMASK = (1 << _CBITS) - 1


_EH = EBLK // 2


def _edge_stage_start(pk2d, vals2d, pb, vb, sems, blk, slot):
    boff = slot * EBLK
    pltpu.async_copy(pk2d.at[blk, pl.ds(0, _EH)],
                     pb.at[pl.ds(boff, _EH)], sems.at[slot, 0])
    pltpu.async_copy(pk2d.at[blk, pl.ds(_EH, _EH)],
                     pb.at[pl.ds(boff + _EH, _EH)], sems.at[slot, 1])
    pltpu.async_copy(vals2d.at[blk, pl.ds(0, _EH)],
                     vb.at[pl.ds(boff, _EH)], sems.at[slot, 2])
    pltpu.async_copy(vals2d.at[blk, pl.ds(_EH, _EH)],
                     vb.at[pl.ds(boff + _EH, _EH)], sems.at[slot, 3])


def _edge_stage_wait(pk2d, vals2d, pb, vb, sems, blk, slot):
    boff = slot * EBLK
    pltpu.make_async_copy(pk2d.at[blk, pl.ds(0, _EH)],
                          pb.at[pl.ds(boff, _EH)], sems.at[slot, 0]).wait()
    pltpu.make_async_copy(pk2d.at[blk, pl.ds(_EH, _EH)],
                          pb.at[pl.ds(boff + _EH, _EH)], sems.at[slot, 1]).wait()
    pltpu.make_async_copy(vals2d.at[blk, pl.ds(0, _EH)],
                          vb.at[pl.ds(boff, _EH)], sems.at[slot, 2]).wait()
    pltpu.make_async_copy(vals2d.at[blk, pl.ds(_EH, _EH)],
                          vb.at[pl.ds(boff + _EH, _EH)], sems.at[slot, 3]).wait()


def _spmm_bn_body(hT, pk2d, vals2d, params, yp_out,
                  h0, h1, a0, a1, ybuf, pb, vb, pv, sems, hsems):
    c = lax.axis_index("c")
    s = lax.axis_index("s")
    wid = s * NC + c
    d0 = wid * 2

    NH = N_PAD // 2
    pltpu.async_copy(hT.at[d0, pl.ds(0, NH)], h0.at[pl.ds(0, NH)],
                     hsems.at[0])
    pltpu.async_copy(hT.at[d0, pl.ds(NH, NH)], h0.at[pl.ds(NH, NH)],
                     hsems.at[1])
    pltpu.async_copy(hT.at[d0 + 1, pl.ds(0, NH)], h1.at[pl.ds(0, NH)],
                     hsems.at[2])
    pltpu.async_copy(hT.at[d0 + 1, pl.ds(NH, NH)], h1.at[pl.ds(NH, NH)],
                     hsems.at[3])
    def blk_of(ci):
        return lax.rem(ci + wid * 4, NBLK)

    _edge_stage_start(pk2d, vals2d, pb, vb, sems, blk_of(0), 0)
    pltpu.sync_copy(params, pv)

    zero = jnp.zeros((L,), jnp.float32)

    @plsc.parallel_loop(0, NVB_PAD)
    def _(j):
        sl = pl.ds(j * L, L)
        a0[sl] = zero
        a1[sl] = zero
        ybuf[sl] = zero

    pltpu.make_async_copy(hT.at[d0, pl.ds(0, NH)], h0.at[pl.ds(0, NH)],
                          hsems.at[0]).wait()
    pltpu.make_async_copy(hT.at[d0, pl.ds(NH, NH)], h0.at[pl.ds(NH, NH)],
                          hsems.at[1]).wait()
    pltpu.make_async_copy(hT.at[d0 + 1, pl.ds(0, NH)], h1.at[pl.ds(0, NH)],
                          hsems.at[2]).wait()
    pltpu.make_async_copy(hT.at[d0 + 1, pl.ds(NH, NH)], h1.at[pl.ds(NH, NH)],
                          hsems.at[3]).wait()

    def chunk_body(ci, carry):
        slot = jnp.bitwise_and(ci, 1)

        @pl.when(ci + 1 < NBLK)
        def _():
            _edge_stage_start(pk2d, vals2d, pb, vb, sems, blk_of(ci + 1),
                              1 - slot)
        _edge_stage_wait(pk2d, vals2d, pb, vb, sems, blk_of(ci), slot)
        boff = slot * EBLK

        @plsc.parallel_loop(0, BPB, unroll=4)
        def _(k):
            sl = pl.ds(boff + k * L, L)
            pk = pb[sl]
            cidx = jnp.bitwise_and(pk, _CMASK)
            ridx = pk >> _CBITS
            v = vb[sl]
            g0 = plsc.load_gather(h0, [cidx]) * v
            plsc.addupdate_scatter(a0, [ridx], g0)
            g1 = plsc.load_gather(h1, [cidx]) * v
            plsc.addupdate_scatter(a1, [ridx], g1)
        return carry
    lax.fori_loop(0, NBLK, chunk_body, 0)

    def stat(acc):
        def sb(j, carry):
            v = acc[pl.ds(j * L, L)]
            return (carry[0] + v, carry[1] + v * v)
        sv, qv = lax.fori_loop(0, NVB, sb, (zero, zero))
        return jnp.sum(sv), jnp.sum(qv)

    inv_n = 1.0 / N_NODES
    s0, q0 = stat(a0)
    s1, q1 = stat(a1)
    m0 = s0 * inv_n
    m1 = s1 * inv_n
    v0 = q0 * inv_n - m0 * m0
    v1 = q1 * inv_n - m1 * m1

    idx0 = jnp.full((L,), d0, jnp.int32)
    idx1 = jnp.full((L,), d0 + 1, jnp.int32)
    gam0 = plsc.load_gather(pv, [idx0])
    gam1 = plsc.load_gather(pv, [idx1])
    bet0 = plsc.load_gather(pv, [idx0 + HID])
    bet1 = plsc.load_gather(pv, [idx1 + HID])
    w20 = plsc.load_gather(pv, [idx0 + 2 * HID])
    w21 = plsc.load_gather(pv, [idx1 + 2 * HID])
    b2v = pv[pl.ds(3 * HID, L)]

    eps = 1e-5
    inv0 = _rsqrt16(jnp.full((L,), v0) + eps) * gam0
    inv1 = _rsqrt16(jnp.full((L,), v1) + eps) * gam1
    sh0 = bet0 - jnp.full((L,), m0) * inv0
    sh1 = bet1 - jnp.full((L,), m1) * inv1
    b2add = b2v * jnp.where(wid == 0, 1.0, 0.0)

    @plsc.parallel_loop(0, NVB, unroll=4)
    def _(j):
        sl = pl.ds(j * L, L)
        t0 = jnp.maximum(a0[sl] * inv0 + sh0, 0.0)
        t1 = jnp.maximum(a1[sl] * inv1 + sh1, 0.0)
        ybuf[sl] = t0 * w20 + t1 * w21 + b2add

    pltpu.sync_copy(ybuf, yp_out.at[wid])


def _spmm_bn(hT, pk2d, vals2d, params):
    mesh = plsc.VectorSubcoreMesh(core_axis_name="c", subcore_axis_name="s")
    f = pl.kernel(
        _spmm_bn_body,
        out_type=jax.ShapeDtypeStruct((NW, N_PAD), jnp.float32),
        mesh=mesh,
        compiler_params=pltpu.CompilerParams(needs_layout_passes=False),
        scratch_types=[
            pltpu.VMEM((N_PAD,), jnp.float32),
            pltpu.VMEM((N_PAD,), jnp.float32),
            pltpu.VMEM((N_PAD,), jnp.float32),
            pltpu.VMEM((N_PAD,), jnp.float32),
            pltpu.VMEM((N_PAD,), jnp.float32),
            pltpu.VMEM((2 * EBLK,), jnp.int32),
            pltpu.VMEM((2 * EBLK,), jnp.float32),
            pltpu.VMEM((256,), jnp.float32),
            pltpu.SemaphoreType.DMA((2, 4)),
            pltpu.SemaphoreType.DMA((4,)),
        ],
    )
    return f(hT, pk2d, vals2d, params)


_RNG = N_PAD // NS


def _spmm2_body(yp, pk2d, vals2d, out_hbm,
                yv, acc, tmp, ys, pb, vb, ysh, osl, sems, rsem):
    c = lax.axis_index("c")
    s = lax.axis_index("s")
    zero = jnp.zeros((L,), jnp.float32)
    nb = _RNG // L

    def reduce_slots(src, nslots, off):
        @plsc.parallel_loop(0, nb)
        def _(j):
            ys[pl.ds(j * L, L)] = zero

        def start(t):
            toff = jnp.bitwise_and(t, 1) * _RNG
            pltpu.async_copy(src.at[t, pl.ds(off, _RNG)],
                             tmp.at[pl.ds(toff, _RNG)], rsem.at[jnp.bitwise_and(t, 1)])

        def wait(t):
            toff = jnp.bitwise_and(t, 1) * _RNG
            pltpu.make_async_copy(src.at[t, pl.ds(off, _RNG)],
                                  tmp.at[pl.ds(toff, _RNG)],
                                  rsem.at[jnp.bitwise_and(t, 1)]).wait()

        start(0)

        def tb(t, carry):
            @pl.when(t + 1 < nslots)
            def _():
                start(t + 1)
            wait(t)
            toff = jnp.bitwise_and(t, 1) * _RNG

            @plsc.parallel_loop(0, nb, unroll=4)
            def _(j):
                sl = pl.ds(j * L, L)
                ys[sl] = ys[sl] + tmp[pl.ds(toff + j * L, L)]
            return carry
        lax.fori_loop(0, nslots, tb, 0)

    @pl.when(c == 0)
    def _():
        off = pl.multiple_of(s * _RNG, 128)
        reduce_slots(yp, NW, off)
        pltpu.sync_copy(ys, ysh.at[pl.ds(off, _RNG)])
        plsc.subcore_barrier()
        pltpu.sync_copy(ysh, yv)

        @plsc.parallel_loop(0, NVB_PAD)
        def _(j):
            acc[pl.ds(j * L, L)] = zero

        nrounds = (NBLK + NS - 1) // NS

        def start_blk(ci, slot):
            _edge_stage_start(pk2d, vals2d, pb, vb, sems, ci * NS + s, slot)

        def wait_blk(ci, slot):
            _edge_stage_wait(pk2d, vals2d, pb, vb, sems, ci * NS + s, slot)

        start_blk(0, 0)

        def chunk_body(ci, carry):
            blk = ci * NS + s
            slot = jnp.bitwise_and(ci, 1)

            @pl.when(blk < NBLK)
            def _():
                @pl.when(blk + NS < NBLK)
                def _():
                    start_blk(ci + 1, 1 - slot)
                wait_blk(ci, slot)
                boff = slot * EBLK

                @plsc.parallel_loop(0, BPB, unroll=4)
                def _(k):
                    sl = pl.ds(boff + k * L, L)
                    pk = pb[sl]
                    g = plsc.load_gather(yv, [jnp.bitwise_and(pk, _CMASK)]) * vb[sl]
                    plsc.addupdate_scatter(acc, [pk >> _CBITS], g)
            return carry
        lax.fori_loop(0, nrounds, chunk_body, 0)

        pltpu.sync_copy(acc, osl.at[s])
        plsc.subcore_barrier()

        reduce_slots(osl, NS, off)
        pltpu.sync_copy(ys, out_hbm.at[pl.ds(off, _RNG)])


def _spmm_scalar(yp, pk2d, vals2d):
    mesh = plsc.VectorSubcoreMesh(core_axis_name="c", subcore_axis_name="s")
    f = pl.kernel(
        _spmm2_body,
        out_type=jax.ShapeDtypeStruct((N_PAD,), jnp.float32),
        mesh=mesh,
        compiler_params=pltpu.CompilerParams(needs_layout_passes=False),
        scratch_types=[
            pltpu.VMEM((N_PAD,), jnp.float32),
            pltpu.VMEM((N_PAD,), jnp.float32),
            pltpu.VMEM((2 * _RNG,), jnp.float32),
            pltpu.VMEM((_RNG,), jnp.float32),
            pltpu.VMEM((2 * EBLK,), jnp.int32),
            pltpu.VMEM((2 * EBLK,), jnp.float32),
            pltpu.VMEM_SHARED((N_PAD,), jnp.float32),
            pltpu.VMEM_SHARED((NS, N_PAD), jnp.float32),
            pltpu.SemaphoreType.DMA((2, 4)),
            pltpu.SemaphoreType.DMA((2,)),
        ],
    )
    return f(yp, pk2d, vals2d)


def kernel(x, A_indices, A_values, W1, b1, gamma, beta, W2, b2):
    rows = A_indices[0].astype(jnp.int32)
    cols = A_indices[1].astype(jnp.int32)
    pk2d = ((rows << _CBITS) | cols).reshape(NBLK, EBLK)
    vals2d = A_values.astype(jnp.float32).reshape(NBLK, EBLK)
    params = jnp.concatenate([
        gamma.astype(jnp.float32),
        beta.astype(jnp.float32),
        W2.reshape(-1).astype(jnp.float32),
        jnp.full((L,), b2[0], jnp.float32),
        jnp.zeros((256 - 3 * HID - L,), jnp.float32),
    ])
    hT = _dense_transposed(x, W1, b1)
    yp = _spmm_bn(hT, pk2d, vals2d, params)
    out = _spmm_scalar(yp, pk2d, vals2d)
    return out[:N_NODES]


# speedup vs baseline: 1.1095x; 1.0835x over previous
import jax
import jax.numpy as jnp
from jax import lax
from jax.experimental import pallas as pl
from jax.experimental.pallas import tpu as pltpu
from jax.experimental.pallas import tpu_sc as plsc

N_NODES = 10000
N_PAD = 10240
N_EDGES = 160000
HID = 64
NPAIR = HID // 2
L = 16
NC = 2
NS = 16
NW = NC * NS
NVB = N_NODES // L
NVB_PAD = N_PAD // L

EBLK = 1280
NBLK = N_EDGES // EBLK
BPB = EBLK // L

_CBITS = 14
_CMASK = (1 << _CBITS) - 1

_NB = 2048
_EB = N_EDGES // (N_PAD // _NB)


def _dense_body(x_ref, w_ref, b_ref, ai_ref, hp_ref, pk_ref):
    acc = lax.dot_general(w_ref[...], x_ref[...], (((1,), (1,)), ((), ())),
                          preferred_element_type=jnp.float32)
    acc = acc + b_ref[...]
    he = lax.bitcast_convert_type(acc[:NPAIR].astype(jnp.bfloat16),
                                  jnp.uint16).astype(jnp.int32)
    ho = lax.bitcast_convert_type(acc[NPAIR:].astype(jnp.bfloat16),
                                  jnp.uint16).astype(jnp.int32)
    hp_ref[...] = (he << 16) | ho
    rows = ai_ref[0:1, :]
    cols = ai_ref[1:2, :]
    pk_ref[...] = ((rows << _CBITS) | cols)[None]


def _dense_packed(x, W1p, b1p, A_indices):
    n, k = x.shape
    grid = N_PAD // _NB
    return pl.pallas_call(
        _dense_body,
        grid=(grid,),
        in_specs=[
            pl.BlockSpec((_NB, k), lambda i: (i, 0)),
            pl.BlockSpec((HID, k), lambda i: (0, 0)),
            pl.BlockSpec((HID, 1), lambda i: (0, 0)),
            pl.BlockSpec((2, _EB), lambda i: (0, i)),
        ],
        out_specs=[
            pl.BlockSpec((NPAIR, _NB), lambda i: (0, i)),
            pl.BlockSpec((1, 1, _EB), lambda i: (i, 0, 0)),
        ],
        out_shape=[
            jax.ShapeDtypeStruct((NPAIR, N_PAD), jnp.int32),
            jax.ShapeDtypeStruct((N_PAD // _NB, 1, _EB), jnp.int32),
        ],
    )(x, W1p, b1p[:, None], A_indices)


def _rsqrt16(x):
    i = plsc.bitcast(x, jnp.int32)
    y = plsc.bitcast(jnp.int32(0x5F3759DF) - (i >> 1), jnp.float32)
    for _ in range(3):
        y = y * (1.5 - 0.5 * x * y * y)
    return y


_EH = EBLK // 2
_EPR = 25


def _edge_stage_start(pk, vals, pb, vb, sems, blk, slot):
    boff = slot * EBLK
    r = blk // _EPR
    coff = pl.multiple_of(lax.rem(blk, _EPR) * EBLK, 128)
    off = pl.multiple_of(blk * EBLK, 128)
    pltpu.async_copy(pk.at[r, 0, pl.ds(coff, _EH)],
                     pb.at[pl.ds(boff, _EH)], sems.at[slot, 0])
    pltpu.async_copy(pk.at[r, 0, pl.ds(coff + _EH, _EH)],
                     pb.at[pl.ds(boff + _EH, _EH)], sems.at[slot, 1])
    pltpu.async_copy(vals.at[pl.ds(off, _EH)],
                     vb.at[pl.ds(boff, _EH)], sems.at[slot, 2])
    pltpu.async_copy(vals.at[pl.ds(off + _EH, _EH)],
                     vb.at[pl.ds(boff + _EH, _EH)], sems.at[slot, 3])


def _edge_stage_wait(pk, vals, pb, vb, sems, blk, slot):
    boff = slot * EBLK
    r = blk // _EPR
    coff = pl.multiple_of(lax.rem(blk, _EPR) * EBLK, 128)
    off = pl.multiple_of(blk * EBLK, 128)
    pltpu.make_async_copy(pk.at[r, 0, pl.ds(coff, _EH)],
                          pb.at[pl.ds(boff, _EH)], sems.at[slot, 0]).wait()
    pltpu.make_async_copy(pk.at[r, 0, pl.ds(coff + _EH, _EH)],
                          pb.at[pl.ds(boff + _EH, _EH)], sems.at[slot, 1]).wait()
    pltpu.make_async_copy(vals.at[pl.ds(off, _EH)],
                          vb.at[pl.ds(boff, _EH)], sems.at[slot, 2]).wait()
    pltpu.make_async_copy(vals.at[pl.ds(off + _EH, _EH)],
                          vb.at[pl.ds(boff + _EH, _EH)], sems.at[slot, 3]).wait()


def _unpack_pair(hv):
    f0 = plsc.bitcast(jnp.bitwise_and(hv, jnp.int32(-65536)), jnp.float32)
    f1 = plsc.bitcast(hv << 16, jnp.float32)
    return f0, f1


def _spmm_bn_body(hp, pk, vals, gamma, beta, w2, b2, yp_out,
                  hv, a0, a1, ybuf, pb, vb, gv, bv, wv, b2v, sems, hsems):
    c = lax.axis_index("c")
    s = lax.axis_index("s")
    wid = s * NC + c
    d0 = wid * 2

    NH = N_PAD // 2
    pltpu.async_copy(hp.at[wid, pl.ds(0, NH)], hv.at[pl.ds(0, NH)],
                     hsems.at[0])
    pltpu.async_copy(hp.at[wid, pl.ds(NH, NH)], hv.at[pl.ds(NH, NH)],
                     hsems.at[1])

    def blk_of(ci):
        return lax.rem(ci + wid * 4, NBLK)

    _edge_stage_start(pk, vals, pb, vb, sems, blk_of(0), 0)
    pltpu.sync_copy(gamma, gv)
    pltpu.sync_copy(beta, bv)
    pltpu.sync_copy(w2.at[0], wv)
    pltpu.sync_copy(b2, b2v)

    zero = jnp.zeros((L,), jnp.float32)

    @plsc.parallel_loop(0, NVB_PAD)
    def _(j):
        sl = pl.ds(j * L, L)
        a0[sl] = zero
        a1[sl] = zero
        ybuf[sl] = zero

    pltpu.make_async_copy(hp.at[wid, pl.ds(0, NH)], hv.at[pl.ds(0, NH)],
                          hsems.at[0]).wait()
    pltpu.make_async_copy(hp.at[wid, pl.ds(NH, NH)], hv.at[pl.ds(NH, NH)],
                          hsems.at[1]).wait()

    def chunk_body(ci, carry):
        slot = jnp.bitwise_and(ci, 1)

        @pl.when(ci + 1 < NBLK)
        def _():
            _edge_stage_start(pk, vals, pb, vb, sems, blk_of(ci + 1),
                              1 - slot)
        _edge_stage_wait(pk, vals, pb, vb, sems, blk_of(ci), slot)
        boff = slot * EBLK

        @plsc.parallel_loop(0, BPB, unroll=4)
        def _(k):
            sl = pl.ds(boff + k * L, L)
            pkv = pb[sl]
            cidx = jnp.bitwise_and(pkv, _CMASK)
            ridx = pkv >> _CBITS
            v = vb[sl]
            f0, f1 = _unpack_pair(plsc.load_gather(hv, [cidx]))
            plsc.addupdate_scatter(a0, [ridx], f0 * v)
            plsc.addupdate_scatter(a1, [ridx], f1 * v)
        return carry
    lax.fori_loop(0, NBLK, chunk_body, 0)

    def stat(acc):
        def sb(j, carry):
            v = acc[pl.ds(j * L, L)]
            return (carry[0] + v, carry[1] + v * v)
        sv, qv = lax.fori_loop(0, NVB, sb, (zero, zero))
        return jnp.sum(sv), jnp.sum(qv)

    inv_n = 1.0 / N_NODES
    s0, q0 = stat(a0)
    s1, q1 = stat(a1)
    m0 = s0 * inv_n
    m1 = s1 * inv_n
    v0 = q0 * inv_n - m0 * m0
    v1 = q1 * inv_n - m1 * m1

    idx0 = jnp.full((L,), d0, jnp.int32)
    idx1 = jnp.full((L,), d0 + 1, jnp.int32)
    gam0 = plsc.load_gather(gv, [idx0])
    gam1 = plsc.load_gather(gv, [idx1])
    bet0 = plsc.load_gather(bv, [idx0])
    bet1 = plsc.load_gather(bv, [idx1])
    w20 = plsc.load_gather(wv, [idx0])
    w21 = plsc.load_gather(wv, [idx1])
    b2b = plsc.load_gather(b2v, [jnp.zeros((L,), jnp.int32)])

    eps = 1e-5
    inv0 = _rsqrt16(jnp.full((L,), v0) + eps) * gam0
    inv1 = _rsqrt16(jnp.full((L,), v1) + eps) * gam1
    sh0 = bet0 - jnp.full((L,), m0) * inv0
    sh1 = bet1 - jnp.full((L,), m1) * inv1
    b2add = b2b * jnp.where(wid == 0, 1.0, 0.0)

    @plsc.parallel_loop(0, NVB, unroll=4)
    def _(j):
        sl = pl.ds(j * L, L)
        t0 = jnp.maximum(a0[sl] * inv0 + sh0, 0.0)
        t1 = jnp.maximum(a1[sl] * inv1 + sh1, 0.0)
        ybuf[sl] = t0 * w20 + t1 * w21 + b2add

    pltpu.sync_copy(ybuf, yp_out.at[wid])


def _spmm_bn(hp, pk, vals, gamma, beta, W2, b2):
    mesh = plsc.VectorSubcoreMesh(core_axis_name="c", subcore_axis_name="s")
    f = pl.kernel(
        _spmm_bn_body,
        out_type=jax.ShapeDtypeStruct((NW, N_PAD), jnp.float32),
        mesh=mesh,
        compiler_params=pltpu.CompilerParams(needs_layout_passes=False),
        scratch_types=[
            pltpu.VMEM((N_PAD,), jnp.int32),
            pltpu.VMEM((N_PAD,), jnp.float32),
            pltpu.VMEM((N_PAD,), jnp.float32),
            pltpu.VMEM((N_PAD,), jnp.float32),
            pltpu.VMEM((2 * EBLK,), jnp.int32),
            pltpu.VMEM((2 * EBLK,), jnp.float32),
            pltpu.VMEM((HID,), jnp.float32),
            pltpu.VMEM((HID,), jnp.float32),
            pltpu.VMEM((HID,), jnp.float32),
            pltpu.VMEM((1,), jnp.float32),
            pltpu.SemaphoreType.DMA((2, 4)),
            pltpu.SemaphoreType.DMA((2,)),
        ],
    )
    return f(hp, pk, vals, gamma, beta, W2, b2)


_RNG = N_PAD // NS


def _spmm2_body(yp, pk, vals, out_hbm,
                yv, acc, tmp, ys, pb, vb, ysh, osl, sems, rsem):
    c = lax.axis_index("c")
    s = lax.axis_index("s")
    zero = jnp.zeros((L,), jnp.float32)
    nb = _RNG // L

    def reduce_slots(src, nslots, off):
        @plsc.parallel_loop(0, nb)
        def _(j):
            ys[pl.ds(j * L, L)] = zero

        def start(t):
            toff = jnp.bitwise_and(t, 1) * _RNG
            pltpu.async_copy(src.at[t, pl.ds(off, _RNG)],
                             tmp.at[pl.ds(toff, _RNG)],
                             rsem.at[jnp.bitwise_and(t, 1)])

        def wait(t):
            toff = jnp.bitwise_and(t, 1) * _RNG
            pltpu.make_async_copy(src.at[t, pl.ds(off, _RNG)],
                                  tmp.at[pl.ds(toff, _RNG)],
                                  rsem.at[jnp.bitwise_and(t, 1)]).wait()

        start(0)

        def tb(t, carry):
            @pl.when(t + 1 < nslots)
            def _():
                start(t + 1)
            wait(t)
            toff = jnp.bitwise_and(t, 1) * _RNG

            @plsc.parallel_loop(0, nb, unroll=4)
            def _(j):
                sl = pl.ds(j * L, L)
                ys[sl] = ys[sl] + tmp[pl.ds(toff + j * L, L)]
            return carry
        lax.fori_loop(0, nslots, tb, 0)

    @pl.when(c == 0)
    def _():
        off = pl.multiple_of(s * _RNG, 128)
        reduce_slots(yp, NW, off)
        pltpu.sync_copy(ys, ysh.at[pl.ds(off, _RNG)])
        plsc.subcore_barrier()
        pltpu.sync_copy(ysh, yv)

        @plsc.parallel_loop(0, NVB_PAD)
        def _(j):
            acc[pl.ds(j * L, L)] = zero

        nrounds = (NBLK + NS - 1) // NS

        def start_blk(ci, slot):
            _edge_stage_start(pk, vals, pb, vb, sems, ci * NS + s, slot)

        def wait_blk(ci, slot):
            _edge_stage_wait(pk, vals, pb, vb, sems, ci * NS + s, slot)

        start_blk(0, 0)

        def chunk_body(ci, carry):
            blk = ci * NS + s
            slot = jnp.bitwise_and(ci, 1)

            @pl.when(blk < NBLK)
            def _():
                @pl.when(blk + NS < NBLK)
                def _():
                    start_blk(ci + 1, 1 - slot)
                wait_blk(ci, slot)
                boff = slot * EBLK

                @plsc.parallel_loop(0, BPB, unroll=4)
                def _(k):
                    sl = pl.ds(boff + k * L, L)
                    pkv = pb[sl]
                    g = plsc.load_gather(yv, [jnp.bitwise_and(pkv, _CMASK)])
                    plsc.addupdate_scatter(acc, [pkv >> _CBITS], g * vb[sl])
            return carry
        lax.fori_loop(0, nrounds, chunk_body, 0)

        pltpu.sync_copy(acc, osl.at[s])
        plsc.subcore_barrier()

        reduce_slots(osl, NS, off)
        pltpu.sync_copy(ys, out_hbm.at[pl.ds(off, _RNG)])


def _spmm_scalar(yp, pk, vals):
    mesh = plsc.VectorSubcoreMesh(core_axis_name="c", subcore_axis_name="s")
    f = pl.kernel(
        _spmm2_body,
        out_type=jax.ShapeDtypeStruct((N_PAD,), jnp.float32),
        mesh=mesh,
        compiler_params=pltpu.CompilerParams(needs_layout_passes=False),
        scratch_types=[
            pltpu.VMEM((N_PAD,), jnp.float32),
            pltpu.VMEM((N_PAD,), jnp.float32),
            pltpu.VMEM((2 * _RNG,), jnp.float32),
            pltpu.VMEM((_RNG,), jnp.float32),
            pltpu.VMEM((2 * EBLK,), jnp.int32),
            pltpu.VMEM((2 * EBLK,), jnp.float32),
            pltpu.VMEM_SHARED((N_PAD,), jnp.float32),
            pltpu.VMEM_SHARED((NS, N_PAD), jnp.float32),
            pltpu.SemaphoreType.DMA((2, 4)),
            pltpu.SemaphoreType.DMA((2,)),
        ],
    )
    return f(yp, pk, vals)


def kernel(x, A_indices, A_values, W1, b1, gamma, beta, W2, b2):
    ai = A_indices.astype(jnp.int32)
    vals = A_values.astype(jnp.float32)
    W1p = jnp.concatenate([W1[0::2], W1[1::2]])
    b1p = jnp.concatenate([b1[0::2], b1[1::2]])
    hp, pk = _dense_packed(x, W1p, b1p, ai)
    yp = _spmm_bn(hp, pk, vals, gamma, beta, W2, b2)
    out = _spmm_scalar(yp, pk, vals)
    return out[:N_NODES]


# speedup vs baseline: 1.3337x; 1.2021x over previous
import jax
import jax.numpy as jnp
from jax import lax
from jax.experimental import pallas as pl
from jax.experimental.pallas import tpu as pltpu
from jax.experimental.pallas import tpu_sc as plsc

N_NODES = 10000
N_PAD = 10240
N_EDGES = 160000
HID = 64
NPAIR = HID // 2
L = 16
NC = 2
NS = 16
NW = NC * NS
NVB = N_NODES // L
NVB_PAD = N_PAD // L

EBLK_A = 16000
NBLK_A = N_EDGES // EBLK_A
EBLK_B = 6400
NBLK_B = N_EDGES // EBLK_B

_CBITS = 14
_CMASK = (1 << _CBITS) - 1

_NB = 2048
_EB = N_EDGES // (N_PAD // _NB)


def _dense_body(x_ref, w_ref, b_ref, ai_ref, hp_ref, pk_ref):
    acc = lax.dot_general(w_ref[...], x_ref[...], (((1,), (1,)), ((), ())),
                          preferred_element_type=jnp.float32)
    acc = acc + b_ref[...]
    he = lax.bitcast_convert_type(acc[:NPAIR].astype(jnp.bfloat16),
                                  jnp.uint16).astype(jnp.int32)
    ho = lax.bitcast_convert_type(acc[NPAIR:].astype(jnp.bfloat16),
                                  jnp.uint16).astype(jnp.int32)
    hp_ref[...] = (he << 16) | ho
    rows = ai_ref[0:1, :]
    cols = ai_ref[1:2, :]
    pk_ref[...] = ((rows << _CBITS) | cols)[None]


def _dense_packed(x, W1p, b1p, A_indices):
    n, k = x.shape
    grid = N_PAD // _NB
    return pl.pallas_call(
        _dense_body,
        grid=(grid,),
        in_specs=[
            pl.BlockSpec((_NB, k), lambda i: (i, 0)),
            pl.BlockSpec((HID, k), lambda i: (0, 0)),
            pl.BlockSpec((HID, 1), lambda i: (0, 0)),
            pl.BlockSpec((2, _EB), lambda i: (0, i)),
        ],
        out_specs=[
            pl.BlockSpec((NPAIR, _NB), lambda i: (0, i)),
            pl.BlockSpec((1, 1, _EB), lambda i: (i, 0, 0)),
        ],
        out_shape=[
            jax.ShapeDtypeStruct((NPAIR, N_PAD), jnp.int32),
            jax.ShapeDtypeStruct((N_PAD // _NB, 1, _EB), jnp.int32),
        ],
    )(x, W1p, b1p[:, None], A_indices)


def _rsqrt16(x):
    i = plsc.bitcast(x, jnp.int32)
    y = plsc.bitcast(jnp.int32(0x5F3759DF) - (i >> 1), jnp.float32)
    for _ in range(3):
        y = y * (1.5 - 0.5 * x * y * y)
    return y


_EPR = 32000


def _edge_stage_start(pk, vals, pb, vb, sems, blk, slot, eblk):
    boff = slot * eblk
    per_row = _EPR // eblk
    r = blk // per_row
    coff = pl.multiple_of(lax.rem(blk, per_row) * eblk, 128)
    off = pl.multiple_of(blk * eblk, 128)
    pltpu.async_copy(pk.at[r, 0, pl.ds(coff, eblk)],
                     pb.at[pl.ds(boff, eblk)], sems.at[slot, 0])
    pltpu.async_copy(vals.at[pl.ds(off, eblk)],
                     vb.at[pl.ds(boff, eblk)], sems.at[slot, 1])


def _edge_stage_wait(pk, vals, pb, vb, sems, blk, slot, eblk):
    boff = slot * eblk
    per_row = _EPR // eblk
    r = blk // per_row
    coff = pl.multiple_of(lax.rem(blk, per_row) * eblk, 128)
    off = pl.multiple_of(blk * eblk, 128)
    pltpu.make_async_copy(pk.at[r, 0, pl.ds(coff, eblk)],
                          pb.at[pl.ds(boff, eblk)], sems.at[slot, 0]).wait()
    pltpu.make_async_copy(vals.at[pl.ds(off, eblk)],
                          vb.at[pl.ds(boff, eblk)], sems.at[slot, 1]).wait()


def _unpack_pair(hv):
    f0 = plsc.bitcast(jnp.bitwise_and(hv, jnp.int32(-65536)), jnp.float32)
    f1 = plsc.bitcast(hv << 16, jnp.float32)
    return f0, f1


def _spmm_bn_body(hp, pk, vals, gamma, beta, w2, b2, yp_out,
                  hv, a0, a1, ybuf, pb, vb, gv, bv, wv, b2v, sems, hsems):
    c = lax.axis_index("c")
    s = lax.axis_index("s")
    wid = s * NC + c
    d0 = wid * 2

    NH = N_PAD // 2
    pltpu.async_copy(hp.at[wid, pl.ds(0, NH)], hv.at[pl.ds(0, NH)],
                     hsems.at[0])
    pltpu.async_copy(hp.at[wid, pl.ds(NH, NH)], hv.at[pl.ds(NH, NH)],
                     hsems.at[1])

    def blk_of(ci):
        return lax.rem(ci + wid, NBLK_A)

    _edge_stage_start(pk, vals, pb, vb, sems, blk_of(0), 0, EBLK_A)
    pltpu.sync_copy(gamma, gv)
    pltpu.sync_copy(beta, bv)
    pltpu.sync_copy(w2.at[0], wv)
    pltpu.sync_copy(b2, b2v)

    zero = jnp.zeros((L,), jnp.float32)

    @plsc.parallel_loop(0, NVB_PAD)
    def _(j):
        sl = pl.ds(j * L, L)
        a0[sl] = zero
        a1[sl] = zero
        ybuf[sl] = zero

    pltpu.make_async_copy(hp.at[wid, pl.ds(0, NH)], hv.at[pl.ds(0, NH)],
                          hsems.at[0]).wait()
    pltpu.make_async_copy(hp.at[wid, pl.ds(NH, NH)], hv.at[pl.ds(NH, NH)],
                          hsems.at[1]).wait()

    def chunk_body(ci, carry):
        slot = jnp.bitwise_and(ci, 1)

        @pl.when(ci + 1 < NBLK_A)
        def _():
            _edge_stage_start(pk, vals, pb, vb, sems, blk_of(ci + 1),
                              1 - slot, EBLK_A)
        _edge_stage_wait(pk, vals, pb, vb, sems, blk_of(ci), slot, EBLK_A)
        boff = slot * EBLK_A

        @plsc.parallel_loop(0, EBLK_A // L, unroll=4)
        def _(k):
            sl = pl.ds(boff + k * L, L)
            pkv = pb[sl]
            cidx = jnp.bitwise_and(pkv, _CMASK)
            ridx = pkv >> _CBITS
            v = vb[sl]
            f0, f1 = _unpack_pair(plsc.load_gather(hv, [cidx]))
            plsc.addupdate_scatter(a0, [ridx], f0 * v)
            plsc.addupdate_scatter(a1, [ridx], f1 * v)
        return carry
    lax.fori_loop(0, NBLK_A, chunk_body, 0)

    def stat(acc):
        def sb(j, carry):
            v = acc[pl.ds(j * L, L)]
            return (carry[0] + v, carry[1] + v * v)
        sv, qv = lax.fori_loop(0, NVB, sb, (zero, zero))
        return jnp.sum(sv), jnp.sum(qv)

    inv_n = 1.0 / N_NODES
    s0, q0 = stat(a0)
    s1, q1 = stat(a1)
    m0 = s0 * inv_n
    m1 = s1 * inv_n
    v0 = q0 * inv_n - m0 * m0
    v1 = q1 * inv_n - m1 * m1

    idx0 = jnp.full((L,), d0, jnp.int32)
    idx1 = jnp.full((L,), d0 + 1, jnp.int32)
    gam0 = plsc.load_gather(gv, [idx0])
    gam1 = plsc.load_gather(gv, [idx1])
    bet0 = plsc.load_gather(bv, [idx0])
    bet1 = plsc.load_gather(bv, [idx1])
    w20 = plsc.load_gather(wv, [idx0])
    w21 = plsc.load_gather(wv, [idx1])
    b2b = plsc.load_gather(b2v, [jnp.zeros((L,), jnp.int32)])

    eps = 1e-5
    inv0 = _rsqrt16(jnp.full((L,), v0) + eps) * gam0
    inv1 = _rsqrt16(jnp.full((L,), v1) + eps) * gam1
    sh0 = bet0 - jnp.full((L,), m0) * inv0
    sh1 = bet1 - jnp.full((L,), m1) * inv1
    b2add = b2b * jnp.where(wid == 0, 1.0, 0.0)

    @plsc.parallel_loop(0, NVB, unroll=4)
    def _(j):
        sl = pl.ds(j * L, L)
        t0 = jnp.maximum(a0[sl] * inv0 + sh0, 0.0)
        t1 = jnp.maximum(a1[sl] * inv1 + sh1, 0.0)
        ybuf[sl] = t0 * w20 + t1 * w21 + b2add

    pltpu.sync_copy(ybuf, yp_out.at[wid])


def _spmm_bn(hp, pk, vals, gamma, beta, W2, b2):
    mesh = plsc.VectorSubcoreMesh(core_axis_name="c", subcore_axis_name="s")
    f = pl.kernel(
        _spmm_bn_body,
        out_type=jax.ShapeDtypeStruct((NW, N_PAD), jnp.float32),
        mesh=mesh,
        compiler_params=pltpu.CompilerParams(needs_layout_passes=False),
        scratch_types=[
            pltpu.VMEM((N_PAD,), jnp.int32),
            pltpu.VMEM((N_PAD,), jnp.float32),
            pltpu.VMEM((N_PAD,), jnp.float32),
            pltpu.VMEM((N_PAD,), jnp.float32),
            pltpu.VMEM((2 * EBLK_A,), jnp.int32),
            pltpu.VMEM((2 * EBLK_A,), jnp.float32),
            pltpu.VMEM((HID,), jnp.float32),
            pltpu.VMEM((HID,), jnp.float32),
            pltpu.VMEM((HID,), jnp.float32),
            pltpu.VMEM((1,), jnp.float32),
            pltpu.SemaphoreType.DMA((2, 4)),
            pltpu.SemaphoreType.DMA((2,)),
        ],
    )
    return f(hp, pk, vals, gamma, beta, W2, b2)


_RNG = N_PAD // NS


def _spmm2_body(yp, pk, vals, out_hbm,
                yv, acc, tmp, ys, pb, vb, ysh, osl, sems, rsem):
    c = lax.axis_index("c")
    s = lax.axis_index("s")
    zero = jnp.zeros((L,), jnp.float32)
    nb = _RNG // L

    def reduce_slots(src, nslots, off):
        @plsc.parallel_loop(0, nb)
        def _(j):
            ys[pl.ds(j * L, L)] = zero

        def start(t):
            toff = jnp.bitwise_and(t, 1) * _RNG
            pltpu.async_copy(src.at[t, pl.ds(off, _RNG)],
                             tmp.at[pl.ds(toff, _RNG)],
                             rsem.at[jnp.bitwise_and(t, 1)])

        def wait(t):
            toff = jnp.bitwise_and(t, 1) * _RNG
            pltpu.make_async_copy(src.at[t, pl.ds(off, _RNG)],
                                  tmp.at[pl.ds(toff, _RNG)],
                                  rsem.at[jnp.bitwise_and(t, 1)]).wait()

        start(0)

        def tb(t, carry):
            @pl.when(t + 1 < nslots)
            def _():
                start(t + 1)
            wait(t)
            toff = jnp.bitwise_and(t, 1) * _RNG

            @plsc.parallel_loop(0, nb, unroll=4)
            def _(j):
                sl = pl.ds(j * L, L)
                ys[sl] = ys[sl] + tmp[pl.ds(toff + j * L, L)]
            return carry
        lax.fori_loop(0, nslots, tb, 0)

    @pl.when(c == 0)
    def _():
        off = pl.multiple_of(s * _RNG, 128)
        reduce_slots(yp, NW, off)
        pltpu.sync_copy(ys, ysh.at[pl.ds(off, _RNG)])
        plsc.subcore_barrier()
        pltpu.sync_copy(ysh, yv)

        @plsc.parallel_loop(0, NVB_PAD)
        def _(j):
            acc[pl.ds(j * L, L)] = zero

        nrounds = (NBLK_B + NS - 1) // NS

        def start_blk(ci, slot):
            _edge_stage_start(pk, vals, pb, vb, sems, ci * NS + s, slot,
                              EBLK_B)

        def wait_blk(ci, slot):
            _edge_stage_wait(pk, vals, pb, vb, sems, ci * NS + s, slot,
                             EBLK_B)

        start_blk(0, 0)

        def chunk_body(ci, carry):
            blk = ci * NS + s
            slot = jnp.bitwise_and(ci, 1)

            @pl.when(blk < NBLK_B)
            def _():
                @pl.when(blk + NS < NBLK_B)
                def _():
                    start_blk(ci + 1, 1 - slot)
                wait_blk(ci, slot)
                boff = slot * EBLK_B

                @plsc.parallel_loop(0, EBLK_B // L, unroll=4)
                def _(k):
                    sl = pl.ds(boff + k * L, L)
                    pkv = pb[sl]
                    g = plsc.load_gather(yv, [jnp.bitwise_and(pkv, _CMASK)])
                    plsc.addupdate_scatter(acc, [pkv >> _CBITS], g * vb[sl])
            return carry
        lax.fori_loop(0, nrounds, chunk_body, 0)

        pltpu.sync_copy(acc, osl.at[s])
        plsc.subcore_barrier()

        reduce_slots(osl, NS, off)
        pltpu.sync_copy(ys, out_hbm.at[pl.ds(off, _RNG)])


def _spmm_scalar(yp, pk, vals):
    mesh = plsc.VectorSubcoreMesh(core_axis_name="c", subcore_axis_name="s")
    f = pl.kernel(
        _spmm2_body,
        out_type=jax.ShapeDtypeStruct((N_PAD,), jnp.float32),
        mesh=mesh,
        compiler_params=pltpu.CompilerParams(needs_layout_passes=False),
        scratch_types=[
            pltpu.VMEM((N_PAD,), jnp.float32),
            pltpu.VMEM((N_PAD,), jnp.float32),
            pltpu.VMEM((2 * _RNG,), jnp.float32),
            pltpu.VMEM((_RNG,), jnp.float32),
            pltpu.VMEM((2 * EBLK_B,), jnp.int32),
            pltpu.VMEM((2 * EBLK_B,), jnp.float32),
            pltpu.VMEM_SHARED((N_PAD,), jnp.float32),
            pltpu.VMEM_SHARED((NS, N_PAD), jnp.float32),
            pltpu.SemaphoreType.DMA((2, 4)),
            pltpu.SemaphoreType.DMA((2,)),
        ],
    )
    return f(yp, pk, vals)


def kernel(x, A_indices, A_values, W1, b1, gamma, beta, W2, b2):
    ai = A_indices.astype(jnp.int32)
    vals = A_values.astype(jnp.float32)
    W1p = jnp.concatenate([W1[0::2], W1[1::2]])
    b1p = jnp.concatenate([b1[0::2], b1[1::2]])
    hp, pk = _dense_packed(x, W1p, b1p, ai)
    yp = _spmm_bn(hp, pk, vals, gamma, beta, W2, b2)
    out = _spmm_scalar(yp, pk, vals)
    return out[:N_NODES]
